# baseline (speedup 1.0000x reference)
import functools

import jax
import jax.numpy as jnp
from jax import lax
from jax.experimental import pallas as pl
from jax.experimental.pallas import tpu as pltpu
from jax.experimental.pallas import tpu_sc as plsc

N, C, H, W = 16, 441, 64, 64
NBA = 21
HW = H * W
TOT = N * HW
NC, NS, L = 2, 16, 16
NWORK = NC * NS
NT = 12
P0 = NT * HW
E = (TOT - P0) // NWORK
HLOC = E // W
G = 4
CH = 128
NBUF = 4
TMAX = E // CH + G - 1
DN = TMAX * CH
RB = 512


def _gather_body(x_hbm, y_hbm, out_hbm, y0_v, y1_v, binv, crankv,
                 cnt_v, lbase_v, rows_d, vals_v, outv, c0_tab, sem):
    wid = lax.axis_index("s") * NC + lax.axis_index("c")
    base = P0 + wid * E
    n = base // HW
    h0 = (base % HW) // W
    lane = lax.iota(jnp.int32, L)

    row0 = pl.multiple_of(n * 2 * H + h0, 8)
    row1 = pl.multiple_of(n * 2 * H + H + h0, 8)
    pltpu.sync_copy(y_hbm.at[pl.ds(row0, HLOC)], y0_v)
    pltpu.sync_copy(y_hbm.at[pl.ds(row1, HLOC)], y1_v)

    def zcnt(i, _):
        cnt_v[pl.ds(i * L, L)] = jnp.zeros((L,), jnp.int32)
        return 0

    lax.fori_loop(0, G, zcnt, 0)

    pad_row = P0 + (base - P0 + E) % (TOT - P0)

    def zrow(i, _):
        rows_d[pl.ds(i * L, L)] = pad_row + (lane & 7)
        return 0

    lax.fori_loop(0, DN // L, zrow, 0, unroll=4)

    def ztab(t, _):
        c0_tab[t] = 0
        return 0

    lax.fori_loop(0, TMAX, ztab, 0)

    def p1(i, _):
        hh = i >> 2
        wsl = pl.ds((i & 3) * L, L)
        y0 = y0_v[hh, wsl]
        y1 = y1_v[hh, wsl]
        b0 = jnp.clip((y0 * float(NBA)).astype(jnp.int32), 0, NBA - 1)
        b1 = jnp.clip((y1 * float(NBA)).astype(jnp.int32), 0, NBA - 1)
        bin_ = b0 * NBA + b1
        g = bin_ >> 7
        idx = g * L + lane
        cr = plsc.load_gather(cnt_v, [idx])
        plsc.store_scatter(cnt_v, [idx], cr + 1)
        binv[pl.ds(i * L, L)] = bin_
        crankv[pl.ds(i * L, L)] = cr
        return 0

    lax.fori_loop(0, E // L, p1, 0)

    def p2(g, carry):
        pbase, t0 = carry
        cvec = cnt_v[pl.ds(g * L, L)]
        csum = plsc.cumsum(cvec)
        hist = jnp.sum(cvec)
        lbase_v[pl.ds(g * L, L)] = csum - cvec + pbase
        nt = (hist + CH - 1) >> 7

        def wr(t, _):
            c0_tab[t] = g * 128
            return 0

        lax.fori_loop(t0, t0 + nt, wr, 0)
        return pbase + nt * CH, t0 + nt

    _, t_used = lax.fori_loop(0, G, p2, (jnp.int32(0), jnp.int32(0)))

    def p3(i, _):
        sl = pl.ds(i * L, L)
        g = binv[sl] >> 7
        lb = plsc.load_gather(lbase_v, [g * L + lane])
        dest = lb + crankv[sl]
        plsc.store_scatter(rows_d, [dest], base + i * L + lane)
        return 0

    lax.fori_loop(0, E // L, p3, 0, unroll=2)

    def fire(t, buf):
        return pltpu.async_copy(
            x_hbm.at[
                rows_d.at[pl.ds(t * CH, CH)],
                pl.ds(pl.multiple_of(c0_tab[t], 128), 128),
            ],
            buf,
            sem,
        )

    def select(t, buf):
        def sel(r, _):
            sl = pl.ds(t * CH + r * L, L)
            kd = rows_d[sl] - base
            m = (kd >= 0) & (kd < E)
            bin_ = plsc.load_gather(binv, [kd], mask=m)
            v = plsc.load_gather(buf, [r * L + lane, bin_ & 127], mask=m)
            plsc.store_scatter(outv, [kd], v, mask=m)
            return 0

        lax.fori_loop(0, CH // L, sel, 0)

    del t_used
    bufs = [vals_v.at[b] for b in range(NBUF)]
    descs = [fire(t, bufs[t % NBUF]) for t in range(NBUF - 1)]
    for t in range(TMAX):
        if t + NBUF - 1 < TMAX:
            descs.append(fire(t + NBUF - 1, bufs[(t + NBUF - 1) % NBUF]))
        descs[t].wait()
        select(t, bufs[t % NBUF])

    pltpu.sync_copy(outv, out_hbm.at[pl.ds(base - P0, E)])


_gather_call = functools.partial(
    pl.kernel,
    mesh=plsc.VectorSubcoreMesh(core_axis_name="c", subcore_axis_name="s"),
    compiler_params=pltpu.CompilerParams(needs_layout_passes=False),
    out_type=jax.ShapeDtypeStruct((TOT - P0,), jnp.float32),
    scratch_types=[
        pltpu.VMEM((HLOC, W), jnp.float32),
        pltpu.VMEM((HLOC, W), jnp.float32),
        pltpu.VMEM((E,), jnp.int32),
        pltpu.VMEM((E,), jnp.int32),
        pltpu.VMEM((G * L,), jnp.int32),
        pltpu.VMEM((G * L,), jnp.int32),
        pltpu.VMEM((DN,), jnp.int32),
        pltpu.VMEM((NBUF, CH, 128), jnp.float32),
        pltpu.VMEM((E,), jnp.float32),
        pltpu.SMEM((TMAX,), jnp.int32),
        pltpu.SemaphoreType.DMA,
    ],
)(_gather_body)


def _dense_body(x_ref, y0_ref, y1_ref, o_ref):
    g = pl.program_id(0)
    b0 = jnp.clip((y0_ref[...] * float(NBA)).astype(jnp.int32), 0, NBA - 1)
    b1 = jnp.clip((y1_ref[...] * float(NBA)).astype(jnp.int32), 0, NBA - 1)
    bins = (b0 * NBA + b1)[:, :, None]
    cid = lax.broadcasted_iota(jnp.int32, (RB // W, W, C), 2)
    v = jnp.sum(jnp.where(cid == bins, x_ref[...], 0.0), axis=2)
    part = jnp.sum(jnp.log(v), axis=0, keepdims=True)

    @pl.when(g == 0)
    def _():
        o_ref[...] = jnp.zeros((1, W), jnp.float32)

    o_ref[...] += part


def _combine_body(v_ref, d_ref, o_ref):
    s = jnp.sum(jnp.log(v_ref[...]), axis=0, keepdims=True)
    o_ref[...] = -(d_ref[...] + s[:, :W] + s[:, W:])


def kernel(x, y):
    x2d = jnp.moveaxis(x, 1, 3).reshape(TOT, C)
    y2d = y.reshape(N * 2 * H, W)
    vals = _gather_call(x2d, y2d)
    dense = pl.pallas_call(
        _dense_body,
        grid=(P0 // RB,),
        in_specs=[
            pl.BlockSpec((RB // W, W, C), lambda g: (g, 0, 0)),
            pl.BlockSpec((RB // W, W), lambda g: (g // 8 * 16 + g % 8, 0)),
            pl.BlockSpec(
                (RB // W, W), lambda g: (g // 8 * 16 + 8 + g % 8, 0)
            ),
        ],
        out_specs=pl.BlockSpec((1, W), lambda g: (0, 0)),
        out_shape=jax.ShapeDtypeStruct((1, W), jnp.float32),
    )(x2d.reshape(TOT // W, W, C), y2d, y2d)
    loss = pl.pallas_call(
        _combine_body,
        out_shape=jax.ShapeDtypeStruct((1, W), jnp.float32),
    )(vals.reshape((TOT - P0) // 128, 128), dense)
    return loss[0]


# speedup vs baseline: 1.8159x; 1.8159x over previous
import functools

import jax
import jax.numpy as jnp
from jax import lax
from jax.experimental import pallas as pl
from jax.experimental.pallas import tpu as pltpu
from jax.experimental.pallas import tpu_sc as plsc

N, C, H, W = 16, 441, 64, 64
NBA = 21
HW = H * W
TOT = N * HW
NC, NS, L = 2, 16, 16
NWORK = NC * NS
E = TOT // NWORK
HLOC = E // W
G = 4
CH = 128
NBUF = 4
TMAX = E // CH + G - 1
DN = TMAX * CH


def _gather_body(x_hbm, y_hbm, out_hbm, y0_v, y1_v, binv, crankv,
                 cnt_v, lbase_v, rows_d, vals_v, outv, c0_tab, sem):
    wid = lax.axis_index("s") * NC + lax.axis_index("c")
    base = wid * E
    n = base // HW
    h0 = (base % HW) // W
    lane = lax.iota(jnp.int32, L)

    row0 = pl.multiple_of(n * 2 * H + h0, 8)
    row1 = pl.multiple_of(n * 2 * H + H + h0, 8)
    pltpu.sync_copy(y_hbm.at[pl.ds(row0, HLOC)], y0_v)
    pltpu.sync_copy(y_hbm.at[pl.ds(row1, HLOC)], y1_v)

    def zcnt(i, _):
        cnt_v[pl.ds(i * L, L)] = jnp.zeros((L,), jnp.int32)
        return 0

    lax.fori_loop(0, G, zcnt, 0)

    pad_row = (base + E) % TOT

    def zrow(i, _):
        rows_d[pl.ds(i * L, L)] = pad_row + (lane & 7)
        return 0

    lax.fori_loop(0, DN // L, zrow, 0, unroll=4)

    def ztab(t, _):
        c0_tab[t] = 0
        return 0

    lax.fori_loop(0, TMAX, ztab, 0)

    def p1(i, _):
        hh = i >> 2
        wsl = pl.ds((i & 3) * L, L)
        y0 = y0_v[hh, wsl]
        y1 = y1_v[hh, wsl]
        b0 = jnp.clip((y0 * float(NBA)).astype(jnp.int32), 0, NBA - 1)
        b1 = jnp.clip((y1 * float(NBA)).astype(jnp.int32), 0, NBA - 1)
        bin_ = b0 * NBA + b1
        g = bin_ >> 7
        idx = g * L + lane
        cr = plsc.load_gather(cnt_v, [idx])
        plsc.store_scatter(cnt_v, [idx], cr + 1)
        binv[pl.ds(i * L, L)] = bin_
        crankv[pl.ds(i * L, L)] = cr
        return 0

    lax.fori_loop(0, E // L, p1, 0)

    def p2(g, carry):
        pbase, t0 = carry
        cvec = cnt_v[pl.ds(g * L, L)]
        csum = plsc.cumsum(cvec)
        hist = jnp.sum(cvec)
        lbase_v[pl.ds(g * L, L)] = csum - cvec + pbase
        nt = (hist + CH - 1) >> 7

        def wr(t, _):
            c0_tab[t] = g * 128
            return 0

        lax.fori_loop(t0, t0 + nt, wr, 0)
        return pbase + nt * CH, t0 + nt

    _, t_used = lax.fori_loop(0, G, p2, (jnp.int32(0), jnp.int32(0)))

    def p3(i, _):
        sl = pl.ds(i * L, L)
        g = binv[sl] >> 7
        lb = plsc.load_gather(lbase_v, [g * L + lane])
        dest = lb + crankv[sl]
        plsc.store_scatter(rows_d, [dest], base + i * L + lane)
        return 0

    lax.fori_loop(0, E // L, p3, 0, unroll=2)

    def fire(t, buf):
        return pltpu.async_copy(
            x_hbm.at[
                rows_d.at[pl.ds(t * CH, CH)],
                pl.ds(pl.multiple_of(c0_tab[t], 128), 128),
            ],
            buf,
            sem,
        )

    def select(t, buf):
        def sel(r, _):
            sl = pl.ds(t * CH + r * L, L)
            kd = rows_d[sl] - base
            m = (kd >= 0) & (kd < E)
            bin_ = plsc.load_gather(binv, [kd], mask=m)
            v = plsc.load_gather(buf, [r * L + lane, bin_ & 127], mask=m)
            plsc.store_scatter(outv, [kd], v, mask=m)
            return 0

        lax.fori_loop(0, CH // L, sel, 0)

    del t_used
    bufs = [vals_v.at[b] for b in range(NBUF)]
    descs = [fire(t, bufs[t % NBUF]) for t in range(NBUF - 1)]
    for t in range(TMAX):
        if t + NBUF - 1 < TMAX:
            descs.append(fire(t + NBUF - 1, bufs[(t + NBUF - 1) % NBUF]))
        descs[t].wait()
        select(t, bufs[t % NBUF])

    pltpu.sync_copy(outv, out_hbm.at[pl.ds(base, E)])


_gather_call = functools.partial(
    pl.kernel,
    mesh=plsc.VectorSubcoreMesh(core_axis_name="c", subcore_axis_name="s"),
    compiler_params=pltpu.CompilerParams(needs_layout_passes=False),
    out_type=jax.ShapeDtypeStruct((TOT,), jnp.float32),
    scratch_types=[
        pltpu.VMEM((HLOC, W), jnp.float32),
        pltpu.VMEM((HLOC, W), jnp.float32),
        pltpu.VMEM((E,), jnp.int32),
        pltpu.VMEM((E,), jnp.int32),
        pltpu.VMEM((G * L,), jnp.int32),
        pltpu.VMEM((G * L,), jnp.int32),
        pltpu.VMEM((DN,), jnp.int32),
        pltpu.VMEM((NBUF, CH, 128), jnp.float32),
        pltpu.VMEM((E,), jnp.float32),
        pltpu.SMEM((TMAX,), jnp.int32),
        pltpu.SemaphoreType.DMA,
    ],
)(_gather_body)


def _loss_body(v_ref, o_ref):
    s = jnp.sum(jnp.log(v_ref[...]), axis=0, keepdims=True)
    o_ref[...] = -(s[:, :W] + s[:, W:])


def kernel(x, y):
    x2d = jnp.moveaxis(x, 1, 3).reshape(TOT, C)
    y2d = y.reshape(N * 2 * H, W)
    vals = _gather_call(x2d, y2d)
    loss = pl.pallas_call(
        _loss_body,
        out_shape=jax.ShapeDtypeStruct((1, W), jnp.float32),
    )(vals.reshape(TOT // 128, 128))
    return loss[0]
